# baseline (speedup 1.0000x reference)
import functools

import jax
import jax.numpy as jnp
from jax import lax
from jax.experimental import pallas as pl
from jax.experimental.pallas import tpu as pltpu

NUM_NODES = 1000000
EMBED_SIZE = 64
BATCH = 16384

_R = 8000
_C = NUM_NODES // _R


def _gather_kernel(sids_ref, pos_ref, starts_ref, table_ref, out_ref):
    c = pl.program_id(0)

    def body(k, _):
        local = sids_ref[k] - c * _R
        p = pos_ref[k]
        out_ref[pl.ds(p, 1), :] = table_ref[pl.ds(local, 1), :]
        return _

    lax.fori_loop(starts_ref[c], starts_ref[c + 1], body, 0)


@jax.jit
def _gather(node_embed_weight, node_ids):
    order = jnp.argsort(node_ids)
    sids = jnp.take(node_ids, order)
    starts = jnp.searchsorted(
        sids, jnp.arange(0, NUM_NODES + _R, _R, dtype=jnp.int32)
    ).astype(jnp.int32)
    grid_spec = pltpu.PrefetchScalarGridSpec(
        num_scalar_prefetch=3,
        grid=(_C,),
        in_specs=[pl.BlockSpec((_R, EMBED_SIZE), lambda c, *_: (c, 0))],
        out_specs=pl.BlockSpec((BATCH, EMBED_SIZE), lambda c, *_: (0, 0)),
    )
    return pl.pallas_call(
        _gather_kernel,
        grid_spec=grid_spec,
        out_shape=jax.ShapeDtypeStruct((BATCH, EMBED_SIZE), jnp.float32),
        compiler_params=pltpu.CompilerParams(
            dimension_semantics=("arbitrary",),
        ),
    )(sids, order.astype(jnp.int32), starts, node_embed_weight)


def kernel(node_ids, node_tids, type_ids, node_embed_weight):
    del node_tids, type_ids
    return _gather(node_embed_weight, node_ids)


# speedup vs baseline: 1.9683x; 1.9683x over previous
import functools

import jax
import jax.numpy as jnp
from jax import lax
from jax.experimental import pallas as pl
from jax.experimental.pallas import tpu as pltpu

NUM_NODES = 1000000
EMBED_SIZE = 64
BATCH = 16384

_K = 8192
_C = -(-NUM_NODES // _K)


def _gather_kernel(sids_ref, pos_ref, starts_ref, tblk_ref, out_ref, rows_ref):
    c = pl.program_id(0)
    rows_ref[...] = tblk_ref[...].T

    def body(k, _):
        local = sids_ref[k] - c * _K
        p = pos_ref[k]
        out_ref[pl.ds(p, 1), :] = rows_ref[pl.ds(local, 1), :]
        return _

    lax.fori_loop(starts_ref[c], starts_ref[c + 1], body, 0)


@jax.jit
def _gather(node_embed_weight, node_ids):
    tableT = node_embed_weight.T
    order = jnp.argsort(node_ids)
    sids = jnp.take(node_ids, order)
    starts = jnp.searchsorted(
        sids, jnp.arange(0, (_C + 1) * _K, _K, dtype=jnp.int32)
    ).astype(jnp.int32)
    grid_spec = pltpu.PrefetchScalarGridSpec(
        num_scalar_prefetch=3,
        grid=(_C,),
        in_specs=[pl.BlockSpec((EMBED_SIZE, _K), lambda c, *_: (0, c))],
        out_specs=pl.BlockSpec((BATCH, EMBED_SIZE), lambda c, *_: (0, 0)),
        scratch_shapes=[pltpu.VMEM((_K, EMBED_SIZE), jnp.float32)],
    )
    return pl.pallas_call(
        _gather_kernel,
        grid_spec=grid_spec,
        out_shape=jax.ShapeDtypeStruct((BATCH, EMBED_SIZE), jnp.float32),
        compiler_params=pltpu.CompilerParams(
            dimension_semantics=("arbitrary",),
        ),
    )(sids, order.astype(jnp.int32), starts, tableT)


def kernel(node_ids, node_tids, type_ids, node_embed_weight):
    del node_tids, type_ids
    return _gather(node_embed_weight, node_ids)


# speedup vs baseline: 2.0741x; 1.0537x over previous
import functools

import jax
import jax.numpy as jnp
from jax import lax
from jax.experimental import pallas as pl
from jax.experimental.pallas import tpu as pltpu

NUM_NODES = 1000000
EMBED_SIZE = 64
BATCH = 16384

_K = 8192
_C = -(-NUM_NODES // _K)


def _gather_kernel(sids_ref, pos_ref, starts_ref, tblk_ref, out_ref, rows_ref):
    c = pl.program_id(0)
    rows_ref[...] = tblk_ref[...].T

    def body(k, _):
        local = sids_ref[k] - c * _K
        p = pos_ref[k]
        out_ref[pl.ds(p, 1), :] = rows_ref[pl.ds(local, 1), :]
        return _

    lax.fori_loop(starts_ref[c], starts_ref[c + 1], body, 0)


@jax.jit
def _gather(node_embed_weight, node_ids):
    tableT = node_embed_weight.T
    order = jnp.argsort(node_ids)
    sids = jnp.take(node_ids, order)
    bounds = jnp.arange(0, (_C + 1) * _K, _K, dtype=jnp.int32)
    starts = jnp.sum(sids[None, :] < bounds[:, None], axis=1, dtype=jnp.int32)
    grid_spec = pltpu.PrefetchScalarGridSpec(
        num_scalar_prefetch=3,
        grid=(_C,),
        in_specs=[pl.BlockSpec((EMBED_SIZE, _K), lambda c, *_: (0, c))],
        out_specs=pl.BlockSpec((BATCH, EMBED_SIZE), lambda c, *_: (0, 0)),
        scratch_shapes=[pltpu.VMEM((_K, EMBED_SIZE), jnp.float32)],
    )
    return pl.pallas_call(
        _gather_kernel,
        grid_spec=grid_spec,
        out_shape=jax.ShapeDtypeStruct((BATCH, EMBED_SIZE), jnp.float32),
        compiler_params=pltpu.CompilerParams(
            dimension_semantics=("arbitrary",),
        ),
    )(sids, order.astype(jnp.int32), starts, tableT)


def kernel(node_ids, node_tids, type_ids, node_embed_weight):
    del node_tids, type_ids
    return _gather(node_embed_weight, node_ids)


# speedup vs baseline: 2.1074x; 1.0161x over previous
import functools

import jax
import jax.numpy as jnp
from jax import lax
from jax.experimental import pallas as pl
from jax.experimental.pallas import tpu as pltpu

NUM_NODES = 1000000
EMBED_SIZE = 64
BATCH = 16384

_K = 32768
_C = -(-NUM_NODES // _K)


def _gather_kernel(sids_ref, pos_ref, starts_ref, tblk_ref, out_ref, rows_ref):
    c = pl.program_id(0)
    rows_ref[...] = tblk_ref[...].T

    def body(k, _):
        local = sids_ref[k] - c * _K
        p = pos_ref[k]
        out_ref[pl.ds(p, 1), :] = rows_ref[pl.ds(local, 1), :]
        return _

    lax.fori_loop(starts_ref[c], starts_ref[c + 1], body, 0)


@jax.jit
def _gather(node_embed_weight, node_ids):
    tableT = node_embed_weight.T
    order = jnp.argsort(node_ids)
    sids = jnp.take(node_ids, order)
    bounds = jnp.arange(0, (_C + 1) * _K, _K, dtype=jnp.int32)
    starts = jnp.sum(sids[None, :] < bounds[:, None], axis=1, dtype=jnp.int32)
    grid_spec = pltpu.PrefetchScalarGridSpec(
        num_scalar_prefetch=3,
        grid=(_C,),
        in_specs=[pl.BlockSpec((EMBED_SIZE, _K), lambda c, *_: (0, c))],
        out_specs=pl.BlockSpec((BATCH, EMBED_SIZE), lambda c, *_: (0, 0)),
        scratch_shapes=[pltpu.VMEM((_K, EMBED_SIZE), jnp.float32)],
    )
    return pl.pallas_call(
        _gather_kernel,
        grid_spec=grid_spec,
        out_shape=jax.ShapeDtypeStruct((BATCH, EMBED_SIZE), jnp.float32),
        compiler_params=pltpu.CompilerParams(
            dimension_semantics=("arbitrary",),
        ),
    )(sids, order.astype(jnp.int32), starts, tableT)


def kernel(node_ids, node_tids, type_ids, node_embed_weight):
    del node_tids, type_ids
    return _gather(node_embed_weight, node_ids)
